# Initial kernel scaffold; baseline (speedup 1.0000x reference)
import jax
import jax.numpy as jnp
from jax.experimental import pallas as pl


def kernel(selected_stem, one_hot_vector, stem_data):
    raise NotImplementedError("write your pallas kernel here")


# speedup vs baseline: 5.8122x; 5.8122x over previous
import functools

import jax
import jax.numpy as jnp
import numpy as np
from jax import lax
from jax.experimental import pallas as pl
from jax.experimental.pallas import tpu as pltpu
from jax.experimental.pallas import tpu_sc as plsc

_GROUP_SIZE = 4
_NC = 2
_NS = 16
_NW = _NC * _NS

_BATCH, _STREAMS, _CH, _T = 32, 4, 2, 44100
_ROWS = _BATCH * _STREAMS * _CH
_SEL_ROWS = _BATCH * _CH


_PERM = (
    (1, 1, 1, 0, 0, 0, 2, 1, 2, 2, 3, 3, 3, 3, 0, 2),
    (1, 1, 1, 2, 2, 0, 0, 3, 3, 2, 2, 1, 0, 3, 3, 0),
    (0, 3, 1, 1, 2, 1, 2, 0, 3, 2, 0, 2, 1, 0, 3, 3),
    (3, 3, 0, 3, 2, 0, 2, 0, 1, 2, 1, 1, 0, 1, 3, 2),
    (0, 0, 2, 0, 3, 3, 3, 1, 1, 2, 1, 3, 2, 1, 0, 2),
    (1, 1, 1, 2, 3, 3, 3, 0, 0, 2, 2, 1, 2, 0, 0, 3),
    (3, 3, 1, 1, 0, 0, 2, 2, 1, 1, 0, 0, 2, 2, 3, 3),
    (0, 3, 2, 0, 2, 2, 0, 1, 1, 1, 3, 2, 3, 0, 1, 3),
)


def _build_index_table() -> np.ndarray:
    perm = np.asarray(_PERM, np.int64).reshape(8, 4, 4)
    perm_flat = perm.reshape(_BATCH, _STREAMS)
    b = np.arange(_BATCH)
    g = b // _GROUP_SIZE
    src_b = g[:, None] * _GROUP_SIZE + perm_flat
    s = np.arange(_STREAMS)
    stem_src = src_b[:, :, None] * (_STREAMS * _CH) + s[None, :, None] * _CH \
        + np.arange(_CH)[None, None, :]
    stem_src = stem_src.reshape(_BATCH, _STREAMS * _CH)
    rowmap = src_b * (_STREAMS * _CH) + s[None, :] * _CH
    table = np.concatenate(
        [stem_src, rowmap, np.zeros((_BATCH, 4), np.int64)], axis=1)
    return table.astype(np.int32)


_IDX_TABLE = _build_index_table()

_N_COPIES = _STREAMS * _CH + _CH


def _sc_body(stem_flat, idx_all, oh_pad, sel_out, stem_out,
             idx_v, oh_v, buf0, buf1, sem_in, sem_out):
    wid = lax.axis_index("s") * _NC + lax.axis_index("c")
    pltpu.sync_copy(idx_all.at[wid], idx_v)
    pltpu.sync_copy(oh_pad.at[wid], oh_v)

    lanes = lax.iota(jnp.int32, 16)
    zeros16 = jnp.zeros((16,), jnp.int32)
    tabv = idx_v[...]

    v = oh_v[...]
    big = jnp.full((16,), 16, jnp.int32)
    p = jnp.min(jnp.where(v == jnp.max(v), lanes, big))

    def table_at(col):
        return jnp.max(jnp.where(lanes == col, tabv, zeros16))

    sel0 = table_at(8 + p)
    srcs = [table_at(j) for j in range(_STREAMS * _CH)] + [sel0, sel0 + 1]
    dsts = ([(stem_out, wid * (_STREAMS * _CH) + j)
             for j in range(_STREAMS * _CH)]
            + [(sel_out, wid * _CH), (sel_out, wid * _CH + 1)])

    bufs = [buf0, buf1]

    def start_gather(j, buf):
        return pltpu.async_copy(
            stem_flat.at[pl.ds(srcs[j], 1)], buf, sem_in)

    def start_scatter(j, buf):
        ref, row = dsts[j]
        return pltpu.async_copy(buf, ref.at[pl.ds(row, 1)], sem_out)

    h_in = start_gather(0, bufs[0])
    h_out = None
    for j in range(_N_COPIES):
        h_in.wait()
        if h_out is not None:
            h_out.wait()
        if j + 1 < _N_COPIES:
            h_in = start_gather(j + 1, bufs[(j + 1) % 2])
        h_out = start_scatter(j, bufs[j % 2])
    h_out.wait()


@functools.partial(jax.jit, static_argnames=())
def _sc_call(stem_flat, idx_all, oh_pad):
    mesh = plsc.VectorSubcoreMesh(core_axis_name="c", subcore_axis_name="s",
                                  num_cores=_NC, num_subcores=_NS)
    return pl.kernel(
        _sc_body,
        out_type=(jax.ShapeDtypeStruct((_SEL_ROWS, _T), jnp.float32),
                  jax.ShapeDtypeStruct((_ROWS, _T), jnp.float32)),
        mesh=mesh,
        scratch_types=[
            pltpu.VMEM((16,), jnp.int32),
            pltpu.VMEM((16,), jnp.float32),
            pltpu.VMEM((1, _T), jnp.float32),
            pltpu.VMEM((1, _T), jnp.float32),
            pltpu.SemaphoreType.DMA,
            pltpu.SemaphoreType.DMA,
        ],
        compiler_params=pltpu.CompilerParams(needs_layout_passes=False),
    )(stem_flat, idx_all, oh_pad)


def kernel(selected_stem, one_hot_vector, stem_data):
    del selected_stem
    stem_flat = stem_data.reshape(_ROWS, _T)
    idx_all = jnp.asarray(_IDX_TABLE)
    oh_pad = jnp.concatenate(
        [one_hot_vector,
         jnp.full((_BATCH, 12), -jnp.inf, jnp.float32)], axis=1)
    sel_flat, stem_out_flat = _sc_call(stem_flat, idx_all, oh_pad)
    selected = sel_flat.reshape(_BATCH, _CH, _T)
    stem_out = stem_out_flat.reshape(_BATCH, _STREAMS, _CH, _T)
    return (selected, one_hot_vector, stem_out)


# speedup vs baseline: 28.2221x; 4.8557x over previous
import functools

import jax
import jax.numpy as jnp
import numpy as np
from jax import lax
from jax.experimental import pallas as pl
from jax.experimental.pallas import tpu as pltpu
from jax.experimental.pallas import tpu_sc as plsc

_GROUP_SIZE = 4
_NC = 2
_NS = 16

_BATCH, _STREAMS, _CH, _T = 32, 4, 2, 44100
_ROWS = _BATCH * _STREAMS
_C0 = 22016
_C1 = _T - _C0

_PERM = (
    (1, 1, 1, 0, 0, 0, 2, 1, 2, 2, 3, 3, 3, 3, 0, 2),
    (1, 1, 1, 2, 2, 0, 0, 3, 3, 2, 2, 1, 0, 3, 3, 0),
    (0, 3, 1, 1, 2, 1, 2, 0, 3, 2, 0, 2, 1, 0, 3, 3),
    (3, 3, 0, 3, 2, 0, 2, 0, 1, 2, 1, 1, 0, 1, 3, 2),
    (0, 0, 2, 0, 3, 3, 3, 1, 1, 2, 1, 3, 2, 1, 0, 2),
    (1, 1, 1, 2, 3, 3, 3, 0, 0, 2, 2, 1, 2, 0, 0, 3),
    (3, 3, 1, 1, 0, 0, 2, 2, 1, 1, 0, 0, 2, 2, 3, 3),
    (0, 3, 2, 0, 2, 2, 0, 1, 1, 1, 3, 2, 3, 0, 1, 3),
)


def _build_index_table() -> np.ndarray:
    perm = np.asarray(_PERM, np.int64).reshape(8, 4, 4)
    perm_flat = perm.reshape(_BATCH, _STREAMS)
    b = np.arange(_BATCH)
    g = b // _GROUP_SIZE
    src_b = g[:, None] * _GROUP_SIZE + perm_flat
    s = np.arange(_STREAMS)
    src_row = src_b * _STREAMS + s[None, :]
    table = np.concatenate(
        [src_row, np.zeros((_BATCH, 4), np.int64),
         src_row, np.zeros((_BATCH, 4), np.int64)], axis=1)
    return table.astype(np.int32)


_IDX_TABLE = _build_index_table()


def _sc_body(stem3, idx_all, oh_pad, sel_out, stem_out,
             idx_v, oh_v, buf0, buf1, sem_in, sem_out):
    wid = lax.axis_index("s") * _NC + lax.axis_index("c")
    pltpu.sync_copy(idx_all.at[wid], idx_v)
    pltpu.sync_copy(oh_pad.at[wid], oh_v)

    lanes = lax.iota(jnp.int32, 16)
    zeros16 = jnp.zeros((16,), jnp.int32)
    tabv = idx_v[...]

    v = oh_v[...]
    big = jnp.full((16,), 16, jnp.int32)
    p = jnp.min(jnp.where(v == jnp.max(v), lanes, big))

    def table_at(col):
        return jnp.max(jnp.where(lanes == col, tabv, zeros16))

    srcs = [table_at(j) for j in range(_STREAMS)] + [table_at(8 + p)]
    dsts = ([(stem_out, wid * _STREAMS + j) for j in range(_STREAMS)]
            + [(sel_out, wid)])
    chunks = [(0, _C0), (_C0, _C1)]
    plan = [(r, d, c) for r, d in zip(srcs, dsts) for c in chunks]

    bufs = [buf0, buf1]

    def start_gather(j, buf):
        src, _, (c, w) = plan[j]
        return pltpu.async_copy(
            stem3.at[src, :, pl.ds(c, w)], buf.at[:, pl.ds(0, w)], sem_in)

    def start_scatter(j, buf):
        _, (ref, row), (c, w) = plan[j]
        return pltpu.async_copy(
            buf.at[:, pl.ds(0, w)], ref.at[row, :, pl.ds(c, w)], sem_out)

    n = len(plan)
    h_in = start_gather(0, bufs[0])
    h_out = None
    for j in range(n):
        h_in.wait()
        if h_out is not None:
            h_out.wait()
        if j + 1 < n:
            h_in = start_gather(j + 1, bufs[(j + 1) % 2])
        h_out = start_scatter(j, bufs[j % 2])
    h_out.wait()


@jax.jit
def _sc_call(stem3, idx_all, oh_pad):
    mesh = plsc.VectorSubcoreMesh(core_axis_name="c", subcore_axis_name="s",
                                  num_cores=_NC, num_subcores=_NS)
    return pl.kernel(
        _sc_body,
        out_type=(jax.ShapeDtypeStruct((_BATCH, _CH, _T), jnp.float32),
                  jax.ShapeDtypeStruct((_ROWS, _CH, _T), jnp.float32)),
        mesh=mesh,
        scratch_types=[
            pltpu.VMEM((16,), jnp.int32),
            pltpu.VMEM((16,), jnp.float32),
            pltpu.VMEM((_CH, _C1), jnp.float32),
            pltpu.VMEM((_CH, _C1), jnp.float32),
            pltpu.SemaphoreType.DMA,
            pltpu.SemaphoreType.DMA,
        ],
        compiler_params=pltpu.CompilerParams(needs_layout_passes=False),
    )(stem3, idx_all, oh_pad)


def kernel(selected_stem, one_hot_vector, stem_data):
    del selected_stem
    stem3 = stem_data.reshape(_ROWS, _CH, _T)
    idx_all = jnp.asarray(_IDX_TABLE)
    oh_pad = jnp.concatenate(
        [one_hot_vector,
         jnp.full((_BATCH, 12), -jnp.inf, jnp.float32)], axis=1)
    selected, stem_out3 = _sc_call(stem3, idx_all, oh_pad)
    stem_out = stem_out3.reshape(_BATCH, _STREAMS, _CH, _T)
    return (selected, one_hot_vector, stem_out)


# speedup vs baseline: 28.8658x; 1.0228x over previous
import functools

import jax
import jax.numpy as jnp
import numpy as np
from jax import lax
from jax.experimental import pallas as pl
from jax.experimental.pallas import tpu as pltpu
from jax.experimental.pallas import tpu_sc as plsc

_GROUP_SIZE = 4
_NC = 2
_NS = 16

_BATCH, _STREAMS, _CH, _T = 32, 4, 2, 44100
_ROWS = _BATCH * _STREAMS
_CW = 11008
_CWT = _T - 3 * _CW
_CHUNKS = ((0, _CW), (_CW, _CW), (2 * _CW, _CW), (3 * _CW, _CWT))
_NBUF = 4

_PERM = (
    (1, 1, 1, 0, 0, 0, 2, 1, 2, 2, 3, 3, 3, 3, 0, 2),
    (1, 1, 1, 2, 2, 0, 0, 3, 3, 2, 2, 1, 0, 3, 3, 0),
    (0, 3, 1, 1, 2, 1, 2, 0, 3, 2, 0, 2, 1, 0, 3, 3),
    (3, 3, 0, 3, 2, 0, 2, 0, 1, 2, 1, 1, 0, 1, 3, 2),
    (0, 0, 2, 0, 3, 3, 3, 1, 1, 2, 1, 3, 2, 1, 0, 2),
    (1, 1, 1, 2, 3, 3, 3, 0, 0, 2, 2, 1, 2, 0, 0, 3),
    (3, 3, 1, 1, 0, 0, 2, 2, 1, 1, 0, 0, 2, 2, 3, 3),
    (0, 3, 2, 0, 2, 2, 0, 1, 1, 1, 3, 2, 3, 0, 1, 3),
)


def _build_index_table() -> np.ndarray:
    perm = np.asarray(_PERM, np.int64).reshape(8, 4, 4)
    perm_flat = perm.reshape(_BATCH, _STREAMS)
    b = np.arange(_BATCH)
    g = b // _GROUP_SIZE
    src_b = g[:, None] * _GROUP_SIZE + perm_flat
    s = np.arange(_STREAMS)
    src_row = src_b * _STREAMS + s[None, :]
    table = np.concatenate(
        [src_row, np.zeros((_BATCH, 4), np.int64),
         src_row, np.zeros((_BATCH, 4), np.int64)], axis=1)
    return table.astype(np.int32)


_IDX_TABLE = _build_index_table()


def _sc_body(stem3, idx_all, oh_pad, sel_out, stem_out,
             idx_v, oh_v, buf0, buf1, buf2, buf3,
             sem_in0, sem_in1, sem_in2, sem_in3,
             sem_out0, sem_out1, sem_out2, sem_out3):
    wid = lax.axis_index("s") * _NC + lax.axis_index("c")
    pltpu.sync_copy(idx_all.at[wid], idx_v)
    pltpu.sync_copy(oh_pad.at[wid], oh_v)

    lanes = lax.iota(jnp.int32, 16)
    zeros16 = jnp.zeros((16,), jnp.int32)
    tabv = idx_v[...]

    v = oh_v[...]
    big = jnp.full((16,), 16, jnp.int32)
    p = jnp.min(jnp.where(v == jnp.max(v), lanes, big))

    def table_at(col):
        return jnp.max(jnp.where(lanes == col, tabv, zeros16))

    srcs = [table_at(j) for j in range(_STREAMS)] + [table_at(8 + p)]
    dsts = ([(stem_out, wid * _STREAMS + j) for j in range(_STREAMS)]
            + [(sel_out, wid)])
    plan = [(r, d, c) for r, d in zip(srcs, dsts) for c in _CHUNKS]

    bufs = [buf0, buf1, buf2, buf3]
    sins = [sem_in0, sem_in1, sem_in2, sem_in3]
    souts = [sem_out0, sem_out1, sem_out2, sem_out3]

    def start_gather(j):
        k = j % _NBUF
        src, _, (c, w) = plan[j]
        return pltpu.async_copy(
            stem3.at[src, :, pl.ds(c, w)], bufs[k].at[:, pl.ds(0, w)],
            sins[k])

    def start_scatter(j):
        k = j % _NBUF
        _, (ref, row), (c, w) = plan[j]
        return pltpu.async_copy(
            bufs[k].at[:, pl.ds(0, w)], ref.at[row, :, pl.ds(c, w)],
            souts[k])

    n = len(plan)
    h_in = [None] * n
    h_out = [None] * n
    for j in range(_NBUF):
        h_in[j] = start_gather(j)
    for j in range(n):
        h_in[j].wait()
        h_out[j] = start_scatter(j)
        if j + _NBUF < n:
            h_out[j].wait()
            h_in[j + _NBUF] = start_gather(j + _NBUF)
    for j in range(n - _NBUF, n):
        h_out[j].wait()


@jax.jit
def _sc_call(stem3, idx_all, oh_pad):
    mesh = plsc.VectorSubcoreMesh(core_axis_name="c", subcore_axis_name="s",
                                  num_cores=_NC, num_subcores=_NS)
    return pl.kernel(
        _sc_body,
        out_type=(jax.ShapeDtypeStruct((_BATCH, _CH, _T), jnp.float32),
                  jax.ShapeDtypeStruct((_ROWS, _CH, _T), jnp.float32)),
        mesh=mesh,
        scratch_types=[
            pltpu.VMEM((16,), jnp.int32),
            pltpu.VMEM((16,), jnp.float32),
            pltpu.VMEM((_CH, _CWT), jnp.float32),
            pltpu.VMEM((_CH, _CWT), jnp.float32),
            pltpu.VMEM((_CH, _CWT), jnp.float32),
            pltpu.VMEM((_CH, _CWT), jnp.float32),
            pltpu.SemaphoreType.DMA,
            pltpu.SemaphoreType.DMA,
            pltpu.SemaphoreType.DMA,
            pltpu.SemaphoreType.DMA,
            pltpu.SemaphoreType.DMA,
            pltpu.SemaphoreType.DMA,
            pltpu.SemaphoreType.DMA,
            pltpu.SemaphoreType.DMA,
        ],
        compiler_params=pltpu.CompilerParams(needs_layout_passes=False),
    )(stem3, idx_all, oh_pad)


def kernel(selected_stem, one_hot_vector, stem_data):
    del selected_stem
    stem3 = stem_data.reshape(_ROWS, _CH, _T)
    idx_all = jnp.asarray(_IDX_TABLE)
    oh_pad = jnp.concatenate(
        [one_hot_vector,
         jnp.full((_BATCH, 12), -jnp.inf, jnp.float32)], axis=1)
    selected, stem_out3 = _sc_call(stem3, idx_all, oh_pad)
    stem_out = stem_out3.reshape(_BATCH, _STREAMS, _CH, _T)
    return (selected, one_hot_vector, stem_out)
